# Initial kernel scaffold; baseline (speedup 1.0000x reference)
import jax
import jax.numpy as jnp
from jax.experimental import pallas as pl


def kernel(x, edge_index, edge_attr, W_l, b_l, W_r, b_r, W_e, att, bias, ln_gamma, ln_beta):
    raise NotImplementedError("write your pallas kernel here")


# speedup vs baseline: 12.2654x; 12.2654x over previous
import functools

import jax
import jax.numpy as jnp
from jax import lax
from jax.experimental import pallas as pl
from jax.experimental.pallas import tpu as pltpu
from jax.experimental.pallas import tpu_sc as plsc

_N = 10000
_E = 320000
_D = 128
_ED = 16
_H = 4
_C = 32
_HC = _H * _C

_NW = 32
_EPW = _E // _NW
_K = 16
_NCHUNK = _EPW // _K
_NZCHUNK = _N // _K


def _proj_body(x_ref, wl_ref, bl_ref, wr_ref, br_ref, xl_ref, xr_ref):
    xb = x_ref[...]
    xl_ref[...] = jnp.dot(xb, wl_ref[...], preferred_element_type=jnp.float32) + bl_ref[0]
    xr_ref[...] = jnp.dot(xb, wr_ref[...], preferred_element_type=jnp.float32) + br_ref[0]


def _proj(x, W_l, b_l, W_r, b_r):
    blk = 1000
    grid = _N // blk
    return pl.pallas_call(
        _proj_body,
        grid=(grid,),
        in_specs=[
            pl.BlockSpec((blk, _D), lambda i: (i, 0)),
            pl.BlockSpec((_D, _HC), lambda i: (0, 0)),
            pl.BlockSpec((1, _HC), lambda i: (0, 0)),
            pl.BlockSpec((_D, _HC), lambda i: (0, 0)),
            pl.BlockSpec((1, _HC), lambda i: (0, 0)),
        ],
        out_specs=[
            pl.BlockSpec((blk, _HC), lambda i: (i, 0)),
            pl.BlockSpec((blk, _HC), lambda i: (i, 0)),
        ],
        out_shape=[
            jax.ShapeDtypeStruct((_N, _HC), jnp.float32),
            jax.ShapeDtypeStruct((_N, _HC), jnp.float32),
        ],
    )(x, W_l, b_l.reshape(1, _HC), W_r, b_r.reshape(1, _HC))


def _eh_body(ea_ref, we_ref, eh_ref):
    eh_ref[...] = jnp.dot(ea_ref[...], we_ref[...], preferred_element_type=jnp.float32)


def _eh(edge_attr, W_e):
    blk = 4000
    grid = _E // blk
    return pl.pallas_call(
        _eh_body,
        grid=(grid,),
        in_specs=[
            pl.BlockSpec((blk, _ED), lambda i: (i, 0)),
            pl.BlockSpec((_ED, _HC), lambda i: (0, 0)),
        ],
        out_specs=pl.BlockSpec((blk, _HC), lambda i: (i, 0)),
        out_shape=jax.ShapeDtypeStruct((_E, _HC), jnp.float32),
    )(edge_attr, W_e)


_DN = _N // 8
_DNP = 1264


def _edge_kernel_body(xl_hbm, xr_hbm, eh_hbm, src_hbm, dst_hbm, att_hbm,
                      acc_hbm, den_hbm,
                      src_v, dst_v, drow_v, xl_v, xr_v, eh_v, msg_v, exb_v,
                      exf_v, q_v, att_v, st_v, acc_sh, den_sh,
                      sem1, sem2, sem3):
    cidx = lax.axis_index("c")
    sid = lax.axis_index("s")
    wid = cidx * 16 + sid

    zero16 = jnp.zeros((16,), jnp.float32)

    def _zero_body(k, carry):
        for j in range(_D // 16):
            st_v[k, pl.ds(16 * j, 16)] = zero16
        exf_v[pl.ds(16 * k, 16)] = zero16
        return carry

    lax.fori_loop(0, 16, _zero_body, 0)

    nz1 = _N // 16
    rem1 = nz1 - (nz1 // 16) * 16
    nt1 = jnp.where(sid < rem1, nz1 // 16 + 1, nz1 // 16)
    nz2 = _DNP // 16
    rem2 = nz2 - (nz2 // 16) * 16
    nt2 = jnp.where(sid < rem2, nz2 // 16 + 1, nz2 // 16)

    def _init1(t, c):
        cid = sid + 16 * t
        pltpu.async_copy(st_v, acc_sh.at[pl.ds(cid * 16, 16)], sem3).wait()
        return c

    lax.fori_loop(0, nt1, _init1, 0)

    def _init2(t, c):
        cid = sid + 16 * t
        pltpu.async_copy(st_v, den_sh.at[pl.ds(cid * 16, 16)], sem3).wait()
        return c

    lax.fori_loop(0, nt2, _init2, 0)

    plsc.subcore_barrier()

    pltpu.async_copy(att_hbm, att_v, sem3).wait()
    attv = [att_v[pl.ds(16 * j, 16)] for j in range(_D // 16)]
    lane = lax.broadcasted_iota(jnp.int32, (16,), 0)

    def _chunk_body(i, carry):
        base = wid * _EPW + i * _K
        pltpu.async_copy(src_hbm.at[pl.ds(base, _K)], src_v, sem3).wait()
        pltpu.async_copy(dst_hbm.at[pl.ds(base, _K)], dst_v, sem3).wait()
        cp1 = pltpu.async_copy(xl_hbm.at[src_v], xl_v, sem1)
        cp2 = pltpu.async_copy(xr_hbm.at[dst_v], xr_v, sem2)
        pltpu.async_copy(eh_hbm.at[pl.ds(base, _K)], eh_v, sem3).wait()
        cp1.wait()
        cp2.wait()

        def _q_body(k, c):
            for h in range(_H):
                p = None
                for jj in range(2):
                    j = 2 * h + jj
                    m = (xl_v[k, pl.ds(16 * j, 16)]
                         + xr_v[k, pl.ds(16 * j, 16)]
                         + eh_v[k, pl.ds(16 * j, 16)])
                    m = jnp.where(m >= 0.0, m, 0.2 * m)
                    pj = m * attv[j]
                    p = pj if p is None else p + pj
                q_v[pl.ds(k * 64 + 16 * h, 16)] = p
            return c

        lax.fori_loop(0, _K, _q_body, 0)

        qbase = lane * 64
        for h in range(_H):
            s = None
            for cc in range(16):
                gv = plsc.load_gather(q_v, [qbase + 16 * h + cc])
                s = gv if s is None else s + gv
            ev = jnp.exp(s)
            plsc.store_scatter(exf_v, [lane * 16 + h], ev)

        dvec = dst_v[...]
        drow_v[...] = dvec // 8
        dmodv = dvec - (dvec // 8) * 8
        for k in range(_K):
            exrow = exf_v[pl.ds(16 * k, 16)]
            exs = [exrow[h] for h in range(_H)]
            dmod = dmodv[k]
            for j in range(_D // 16):
                msg_v[k, pl.ds(16 * j, 16)] = xl_v[k, pl.ds(16 * j, 16)] * exs[j // 2]
                exb_v[k, pl.ds(16 * j, 16)] = jnp.where(dmod == j, exrow, zero16)

        pltpu.async_copy(msg_v, acc_sh.at[dst_v], sem1, add=True).wait()
        pltpu.async_copy(exb_v, den_sh.at[drow_v], sem2, add=True).wait()
        return carry

    lax.fori_loop(0, _NCHUNK, _chunk_body, 0)

    plsc.subcore_barrier()

    def _out1(t, c):
        cid = sid + 16 * t
        pltpu.async_copy(acc_sh.at[pl.ds(cid * 16, 16)], st_v, sem3).wait()
        pltpu.async_copy(st_v, acc_hbm.at[cidx, pl.ds(cid * 16, 16)], sem3).wait()
        return c

    lax.fori_loop(0, nt1, _out1, 0)

    def _out2(t, c):
        cid = sid + 16 * t
        pltpu.async_copy(den_sh.at[pl.ds(cid * 16, 16)], st_v, sem3).wait()
        pltpu.async_copy(st_v, den_hbm.at[cidx, pl.ds(cid * 16, 16)], sem3).wait()
        return c

    lax.fori_loop(0, nt2, _out2, 0)
    plsc.subcore_barrier()


def _edge_pass(xl, xr, eh, src, dst, att_flat):
    mesh = plsc.VectorSubcoreMesh(core_axis_name="c", subcore_axis_name="s")
    f = functools.partial(
        pl.kernel,
        mesh=mesh,
        compiler_params=pltpu.CompilerParams(needs_layout_passes=False),
        out_type=[
            jax.ShapeDtypeStruct((2, _N, _D), jnp.float32),
            jax.ShapeDtypeStruct((2, _DNP, _D), jnp.float32),
        ],
        scratch_types=[
            pltpu.VMEM((_K,), jnp.int32),
            pltpu.VMEM((_K,), jnp.int32),
            pltpu.VMEM((_K,), jnp.int32),
            pltpu.VMEM((_K, _D), jnp.float32),
            pltpu.VMEM((_K, _D), jnp.float32),
            pltpu.VMEM((_K, _D), jnp.float32),
            pltpu.VMEM((_K, _D), jnp.float32),
            pltpu.VMEM((_K, _D), jnp.float32),
            pltpu.VMEM((_K * 16,), jnp.float32),
            pltpu.VMEM((_K * 64,), jnp.float32),
            pltpu.VMEM((_D,), jnp.float32),
            pltpu.VMEM((16, _D), jnp.float32),
            pltpu.VMEM_SHARED((_N, _D), jnp.float32),
            pltpu.VMEM_SHARED((_DNP, _D), jnp.float32),
            pltpu.SemaphoreType.DMA,
            pltpu.SemaphoreType.DMA,
            pltpu.SemaphoreType.DMA,
        ],
    )(_edge_kernel_body)
    return f(xl, xr, eh, src, dst, att_flat)


def _epi_body(acc_ref, den_ref, x_ref, bmat_ref, bias_ref, g_ref, b_ref, y_ref):
    num = acc_ref[0] + acc_ref[1]
    den = den_ref[0] + den_ref[1]
    denw = jnp.dot(den, bmat_ref[...], preferred_element_type=jnp.float32)
    o = num / (denw + 1e-16) + bias_ref[0]
    h = o * jax.nn.sigmoid(o)
    y = h + x_ref[...]
    mu = jnp.mean(y, axis=-1, keepdims=True)
    d = y - mu
    var = jnp.mean(d * d, axis=-1, keepdims=True)
    yn = d * lax.rsqrt(var + 1e-5)
    y_ref[...] = yn * g_ref[0] + b_ref[0]


def _epilogue(acc, den, x, bias, ln_gamma, ln_beta):
    blk = 1000
    grid = _N // blk
    rows = jnp.arange(16, dtype=jnp.int32)[:, None]
    cols = jnp.arange(_HC, dtype=jnp.int32)[None, :]
    bmat = (rows == cols // _C).astype(jnp.float32)
    return pl.pallas_call(
        _epi_body,
        grid=(grid,),
        in_specs=[
            pl.BlockSpec((2, blk, _D), lambda i: (0, i, 0)),
            pl.BlockSpec((2, blk, 16), lambda i: (0, i, 0)),
            pl.BlockSpec((blk, _D), lambda i: (i, 0)),
            pl.BlockSpec((16, _HC), lambda i: (0, 0)),
            pl.BlockSpec((1, _HC), lambda i: (0, 0)),
            pl.BlockSpec((1, _HC), lambda i: (0, 0)),
            pl.BlockSpec((1, _HC), lambda i: (0, 0)),
        ],
        out_specs=pl.BlockSpec((blk, _HC), lambda i: (i, 0)),
        out_shape=jax.ShapeDtypeStruct((_N, _HC), jnp.float32),
    )(acc, den, x, bmat, bias.reshape(1, _HC), ln_gamma.reshape(1, _HC),
      ln_beta.reshape(1, _HC))


def kernel(x, edge_index, edge_attr, W_l, b_l, W_r, b_r, W_e, att, bias,
           ln_gamma, ln_beta):
    src = edge_index[0]
    dst = edge_index[1]
    xl, xr = _proj(x, W_l, b_l, W_r, b_r)
    eh = _eh(edge_attr, W_e)
    acc, den_packed = _edge_pass(xl, xr, eh, src, dst, att.reshape(_HC))
    den = den_packed[:, :_DN, :].reshape(2, _N, 16)
    return _epilogue(acc, den, x, bias, ln_gamma, ln_beta)


# speedup vs baseline: 16.6509x; 1.3575x over previous
import functools

import jax
import jax.numpy as jnp
from jax import lax
from jax.experimental import pallas as pl
from jax.experimental.pallas import tpu as pltpu
from jax.experimental.pallas import tpu_sc as plsc

_N = 10000
_E = 320000
_D = 128
_ED = 16
_H = 4
_C = 32
_HC = _H * _C

_NW = 32
_EPW = _E // _NW
_K = 16
_NCHUNK = _EPW // _K
_NZCHUNK = _N // _K


def _proj_body(x_ref, wl_ref, bl_ref, wr_ref, br_ref, xl_ref, xr_ref):
    xb = x_ref[...]
    xl_ref[...] = jnp.dot(xb, wl_ref[...], preferred_element_type=jnp.float32) + bl_ref[0]
    xr_ref[...] = jnp.dot(xb, wr_ref[...], preferred_element_type=jnp.float32) + br_ref[0]


def _proj(x, W_l, b_l, W_r, b_r):
    blk = 1000
    grid = _N // blk
    return pl.pallas_call(
        _proj_body,
        grid=(grid,),
        in_specs=[
            pl.BlockSpec((blk, _D), lambda i: (i, 0)),
            pl.BlockSpec((_D, _HC), lambda i: (0, 0)),
            pl.BlockSpec((1, _HC), lambda i: (0, 0)),
            pl.BlockSpec((_D, _HC), lambda i: (0, 0)),
            pl.BlockSpec((1, _HC), lambda i: (0, 0)),
        ],
        out_specs=[
            pl.BlockSpec((blk, _HC), lambda i: (i, 0)),
            pl.BlockSpec((blk, _HC), lambda i: (i, 0)),
        ],
        out_shape=[
            jax.ShapeDtypeStruct((_N, _HC), jnp.float32),
            jax.ShapeDtypeStruct((_N, _HC), jnp.float32),
        ],
    )(x, W_l, b_l.reshape(1, _HC), W_r, b_r.reshape(1, _HC))


def _eh_body(ea_ref, we_ref, eh_ref):
    eh_ref[...] = jnp.dot(ea_ref[...], we_ref[...], preferred_element_type=jnp.float32)


def _eh(edge_attr, W_e):
    blk = 4000
    grid = _E // blk
    return pl.pallas_call(
        _eh_body,
        grid=(grid,),
        in_specs=[
            pl.BlockSpec((blk, _ED), lambda i: (i, 0)),
            pl.BlockSpec((_ED, _HC), lambda i: (0, 0)),
        ],
        out_specs=pl.BlockSpec((blk, _HC), lambda i: (i, 0)),
        out_shape=jax.ShapeDtypeStruct((_E, _HC), jnp.float32),
    )(edge_attr, W_e)


_DN = _N // 8
_DNP = 1264


def _edge_kernel_body(xl_hbm, xr_hbm, eh_hbm, src_hbm, dst_hbm, att_hbm,
                      acc_hbm, den_hbm,
                      src_v, dst_v, drow_v, xl_v, xr_v, eh_v, msg_v, exb_v,
                      exf_v, q_v,
                      src2_v, dst2_v, drow2_v, xl2_v, xr2_v, eh2_v, msg2_v,
                      exb2_v, exf2_v, q2_v,
                      att_v, st_v, acc_sh, den_sh,
                      sem1, sem2, sem3, sem4, sem5, sem6):
    cidx = lax.axis_index("c")
    sid = lax.axis_index("s")
    wid = cidx * 16 + sid

    zero16 = jnp.zeros((16,), jnp.float32)

    def _zero_body(k, carry):
        for j in range(_D // 16):
            st_v[k, pl.ds(16 * j, 16)] = zero16
        exf_v[pl.ds(16 * k, 16)] = zero16
        exf2_v[pl.ds(16 * k, 16)] = zero16
        return carry

    lax.fori_loop(0, 16, _zero_body, 0)

    nz1 = _N // 16
    rem1 = nz1 - (nz1 // 16) * 16
    nt1 = jnp.where(sid < rem1, nz1 // 16 + 1, nz1 // 16)
    nz2 = _DNP // 16
    rem2 = nz2 - (nz2 // 16) * 16
    nt2 = jnp.where(sid < rem2, nz2 // 16 + 1, nz2 // 16)

    def _init1(t, c):
        cid = sid + 16 * t
        pltpu.async_copy(st_v, acc_sh.at[pl.ds(cid * 16, 16)], sem3).wait()
        return c

    lax.fori_loop(0, nt1, _init1, 0)

    def _init2(t, c):
        cid = sid + 16 * t
        pltpu.async_copy(st_v, den_sh.at[pl.ds(cid * 16, 16)], sem3).wait()
        return c

    lax.fori_loop(0, nt2, _init2, 0)

    plsc.subcore_barrier()

    pltpu.async_copy(att_hbm, att_v, sem3).wait()
    attv = [att_v[pl.ds(16 * j, 16)] for j in range(_D // 16)]
    lane = lax.broadcasted_iota(jnp.int32, (16,), 0)

    src_b = [src_v, src2_v]
    dst_b = [dst_v, dst2_v]
    drow_b = [drow_v, drow2_v]
    xl_b = [xl_v, xl2_v]
    xr_b = [xr_v, xr2_v]
    eh_b = [eh_v, eh2_v]
    msg_b = [msg_v, msg2_v]
    exb_b = [exb_v, exb2_v]
    q_b = [q_v, q2_v]
    exf_b = [exf_v, exf2_v]
    sa = [sem1, sem4]
    sb = [sem2, sem5]
    sc = [sem3, sem6]

    def _compute(b):
        xlv_, xrv_, ehv_, msgv_, exbv_ = xl_b[b], xr_b[b], eh_b[b], msg_b[b], exb_b[b]
        qv_, exfv_ = q_b[b], exf_b[b]

        def _q_body(k, c):
            for h in range(_H):
                p = None
                for jj in range(2):
                    j = 2 * h + jj
                    m = (xlv_[k, pl.ds(16 * j, 16)]
                         + xrv_[k, pl.ds(16 * j, 16)]
                         + ehv_[k, pl.ds(16 * j, 16)])
                    m = jnp.where(m >= 0.0, m, 0.2 * m)
                    pj = m * attv[j]
                    p = pj if p is None else p + pj
                qv_[pl.ds(k * 64 + 16 * h, 16)] = p
            return c

        lax.fori_loop(0, _K, _q_body, 0)

        qbase = lane * 64
        for h in range(_H):
            s = None
            for cc in range(16):
                gv = plsc.load_gather(qv_, [qbase + 16 * h + cc])
                s = gv if s is None else s + gv
            ev = jnp.exp(s)
            plsc.store_scatter(exfv_, [lane * 16 + h], ev)

        dvec = dst_b[b][...]
        drow_b[b][...] = dvec // 8
        dmodv = dvec - (dvec // 8) * 8
        for k in range(_K):
            exrow = exfv_[pl.ds(16 * k, 16)]
            exs = [exrow[h] for h in range(_H)]
            dmod = dmodv[k]
            for j in range(_D // 16):
                msgv_[k, pl.ds(16 * j, 16)] = xlv_[k, pl.ds(16 * j, 16)] * exs[j // 2]
                exbv_[k, pl.ds(16 * j, 16)] = jnp.where(dmod == j, exrow, zero16)

    def _run_chunks(bases):
        nb = len(bases)
        cps = []
        for b in range(nb):
            cps.append(pltpu.async_copy(src_hbm.at[pl.ds(bases[b], _K)], src_b[b], sa[b]))
            cps.append(pltpu.async_copy(dst_hbm.at[pl.ds(bases[b], _K)], dst_b[b], sb[b]))
        for cp in cps:
            cp.wait()
        cps = []
        for b in range(nb):
            cps.append(pltpu.async_copy(xl_hbm.at[src_b[b]], xl_b[b], sa[b]))
            cps.append(pltpu.async_copy(xr_hbm.at[dst_b[b]], xr_b[b], sb[b]))
            cps.append(pltpu.async_copy(eh_hbm.at[pl.ds(bases[b], _K)], eh_b[b], sc[b]))
        for cp in cps:
            cp.wait()
        for b in range(nb):
            _compute(b)
        cps = []
        for b in range(nb):
            cps.append(pltpu.async_copy(msg_b[b], acc_sh.at[dst_b[b]], sa[b], add=True))
            cps.append(pltpu.async_copy(exb_b[b], den_sh.at[drow_b[b]], sb[b], add=True))
        for cp in cps:
            cp.wait()

    npairs = _NCHUNK // 2

    def _chunk_body(i, carry):
        base = wid * _EPW + i * (2 * _K)
        _run_chunks([base, base + _K])
        return carry

    lax.fori_loop(0, npairs, _chunk_body, 0)
    if _NCHUNK % 2:
        _run_chunks([wid * _EPW + (_NCHUNK - 1) * _K])

    plsc.subcore_barrier()

    def _out1(t, c):
        cid = sid + 16 * t
        pltpu.async_copy(acc_sh.at[pl.ds(cid * 16, 16)], st_v, sem3).wait()
        pltpu.async_copy(st_v, acc_hbm.at[cidx, pl.ds(cid * 16, 16)], sem3).wait()
        return c

    lax.fori_loop(0, nt1, _out1, 0)

    def _out2(t, c):
        cid = sid + 16 * t
        pltpu.async_copy(den_sh.at[pl.ds(cid * 16, 16)], st_v, sem3).wait()
        pltpu.async_copy(st_v, den_hbm.at[cidx, pl.ds(cid * 16, 16)], sem3).wait()
        return c

    lax.fori_loop(0, nt2, _out2, 0)
    plsc.subcore_barrier()


def _edge_pass(xl, xr, eh, src, dst, att_flat):
    mesh = plsc.VectorSubcoreMesh(core_axis_name="c", subcore_axis_name="s")
    f = functools.partial(
        pl.kernel,
        mesh=mesh,
        compiler_params=pltpu.CompilerParams(needs_layout_passes=False),
        out_type=[
            jax.ShapeDtypeStruct((2, _N, _D), jnp.float32),
            jax.ShapeDtypeStruct((2, _DNP, _D), jnp.float32),
        ],
        scratch_types=(
            [
                pltpu.VMEM((_K,), jnp.int32),
                pltpu.VMEM((_K,), jnp.int32),
                pltpu.VMEM((_K,), jnp.int32),
                pltpu.VMEM((_K, _D), jnp.float32),
                pltpu.VMEM((_K, _D), jnp.float32),
                pltpu.VMEM((_K, _D), jnp.float32),
                pltpu.VMEM((_K, _D), jnp.float32),
                pltpu.VMEM((_K, _D), jnp.float32),
                pltpu.VMEM((_K * 16,), jnp.float32),
                pltpu.VMEM((_K * 64,), jnp.float32),
            ] * 2
            + [
                pltpu.VMEM((_D,), jnp.float32),
                pltpu.VMEM((16, _D), jnp.float32),
                pltpu.VMEM_SHARED((_N, _D), jnp.float32),
                pltpu.VMEM_SHARED((_DNP, _D), jnp.float32),
            ]
            + [pltpu.SemaphoreType.DMA] * 6
        ),
    )(_edge_kernel_body)
    return f(xl, xr, eh, src, dst, att_flat)


def _epi_body(acc_ref, den_ref, x_ref, bmat_ref, bias_ref, g_ref, b_ref, y_ref):
    num = acc_ref[0] + acc_ref[1]
    den = den_ref[0] + den_ref[1]
    denw = jnp.dot(den, bmat_ref[...], preferred_element_type=jnp.float32)
    o = num / (denw + 1e-16) + bias_ref[0]
    h = o * jax.nn.sigmoid(o)
    y = h + x_ref[...]
    mu = jnp.mean(y, axis=-1, keepdims=True)
    d = y - mu
    var = jnp.mean(d * d, axis=-1, keepdims=True)
    yn = d * lax.rsqrt(var + 1e-5)
    y_ref[...] = yn * g_ref[0] + b_ref[0]


def _epilogue(acc, den, x, bias, ln_gamma, ln_beta):
    blk = 1000
    grid = _N // blk
    rows = jnp.arange(16, dtype=jnp.int32)[:, None]
    cols = jnp.arange(_HC, dtype=jnp.int32)[None, :]
    bmat = (rows == cols // _C).astype(jnp.float32)
    return pl.pallas_call(
        _epi_body,
        grid=(grid,),
        in_specs=[
            pl.BlockSpec((2, blk, _D), lambda i: (0, i, 0)),
            pl.BlockSpec((2, blk, 16), lambda i: (0, i, 0)),
            pl.BlockSpec((blk, _D), lambda i: (i, 0)),
            pl.BlockSpec((16, _HC), lambda i: (0, 0)),
            pl.BlockSpec((1, _HC), lambda i: (0, 0)),
            pl.BlockSpec((1, _HC), lambda i: (0, 0)),
            pl.BlockSpec((1, _HC), lambda i: (0, 0)),
        ],
        out_specs=pl.BlockSpec((blk, _HC), lambda i: (i, 0)),
        out_shape=jax.ShapeDtypeStruct((_N, _HC), jnp.float32),
    )(acc, den, x, bmat, bias.reshape(1, _HC), ln_gamma.reshape(1, _HC),
      ln_beta.reshape(1, _HC))


def kernel(x, edge_index, edge_attr, W_l, b_l, W_r, b_r, W_e, att, bias,
           ln_gamma, ln_beta):
    src = edge_index[0]
    dst = edge_index[1]
    xl, xr = _proj(x, W_l, b_l, W_r, b_r)
    eh = _eh(edge_attr, W_e)
    acc, den_packed = _edge_pass(xl, xr, eh, src, dst, att.reshape(_HC))
    den = den_packed[:, :_DN, :].reshape(2, _N, 16)
    return _epilogue(acc, den, x, bias, ln_gamma, ln_beta)
